# Initial kernel scaffold; baseline (speedup 1.0000x reference)
import jax
import jax.numpy as jnp
from jax.experimental import pallas as pl


def kernel(features, node, neighbours, attention_weights, kernel, kernel1, neigh_weights):
    raise NotImplementedError("write your pallas kernel here")


# speedup vs baseline: 4.6821x; 4.6821x over previous
import functools

import jax
import jax.numpy as jnp
from jax import lax
from jax.experimental import pallas as pl
from jax.experimental.pallas import tpu as pltpu
from jax.experimental.pallas import tpu_sc as plsc

N_NODES = 100000
D = 128
B = 8192
K = 32

_F32 = jnp.float32


_PQ_ROWS = 2000


def _pq_body(f_ref, k0_ref, k1_ref, aw_ref, p_ref, q_ref):
    awn = aw_ref[0, :D].reshape(D, 1)
    awt = aw_ref[0, D:].reshape(D, 1)
    v1 = jnp.dot(k1_ref[...], awn, preferred_element_type=_F32)
    v2 = jnp.dot(k0_ref[...], awt, preferred_element_type=_F32)
    f = f_ref[...]
    p_ref[...] = jnp.dot(f, v1, preferred_element_type=_F32)
    q_ref[...] = jnp.dot(f, v2, preferred_element_type=_F32)


def _pq_pass(features, k0, k1, aw):
    return pl.pallas_call(
        _pq_body,
        grid=(N_NODES // _PQ_ROWS,),
        in_specs=[
            pl.BlockSpec((_PQ_ROWS, D), lambda i: (i, 0)),
            pl.BlockSpec((D, D), lambda i: (0, 0)),
            pl.BlockSpec((D, D), lambda i: (0, 0)),
            pl.BlockSpec((1, 2 * D), lambda i: (0, 0)),
        ],
        out_specs=[
            pl.BlockSpec((_PQ_ROWS, 1), lambda i: (i, 0)),
            pl.BlockSpec((_PQ_ROWS, 1), lambda i: (i, 0)),
        ],
        out_shape=[
            jax.ShapeDtypeStruct((N_NODES, 1), _F32),
            jax.ShapeDtypeStruct((N_NODES, 1), _F32),
        ],
    )(features, k0, k1, aw)


_NW = 32
_BPW = B // _NW
_IPW = _BPW * K
_NB = 8
_RB = _NB * K
_NBLK = _BPW // _NB
_C = D // 16


def _sc_body(feat, p_hbm, q_hbm, nbr_hbm, node_hbm, agg_hbm,
             idx_v, s_v, nidx_v, qv_v, rows_v, agg_v, sem, sem2):
    nc = plsc.get_sparse_core_info().num_cores
    wid = lax.axis_index("s") * nc + lax.axis_index("c")
    ibase = wid * _IPW
    nbase = wid * _BPW

    pltpu.sync_copy(nbr_hbm.at[pl.ds(ibase, _IPW)], idx_v)
    pltpu.sync_copy(node_hbm.at[pl.ds(nbase, _BPW)], nidx_v)
    pltpu.async_copy(p_hbm.at[idx_v], s_v, sem).wait()
    pltpu.async_copy(q_hbm.at[nidx_v], qv_v, sem2).wait()

    def wbody(g, carry):
        qv = qv_v[pl.ds(g * 16, 16)]
        for j in range(16):
            base = (g * 16 + j) * K
            qb = qv[j]
            a0 = s_v[pl.ds(base, 16)] + qb
            a1 = s_v[pl.ds(base + 16, 16)] + qb
            a0 = jnp.where(a0 >= 0.0, a0, a0 * 0.2)
            a1 = jnp.where(a1 >= 0.0, a1, a1 * 0.2)
            m = jnp.maximum(jnp.max(a0), jnp.max(a1))
            e0 = jnp.exp(a0 - m)
            e1 = jnp.exp(a1 - m)
            den = jnp.broadcast_to(jnp.sum(e0) + jnp.sum(e1), (16,))
            s_v[pl.ds(base, 16)] = e0 / den
            s_v[pl.ds(base + 16, 16)] = e1 / den
        return carry

    lax.fori_loop(0, _BPW // 16, wbody, 0)

    def blkbody(blk, carry):
        pltpu.async_copy(
            feat.at[idx_v.at[pl.ds(blk * _RB, _RB)]], rows_v, sem).wait()

        def nbody(j, carry2):
            b0 = (blk * _NB + j) * K
            w0 = s_v[pl.ds(b0, 16)]
            w1 = s_v[pl.ds(b0 + 16, 16)]
            accs = tuple(jnp.zeros((16,), _F32) for _ in range(_C))
            for k in range(K):
                wk = w0[k] if k < 16 else w1[k - 16]
                r = j * K + k
                accs = tuple(
                    accs[c] + wk * rows_v[r, pl.ds(c * 16, 16)]
                    for c in range(_C))
            for c in range(_C):
                agg_v[j, pl.ds(c * 16, 16)] = accs[c]
            return carry2

        lax.fori_loop(0, _NB, nbody, 0)
        pltpu.sync_copy(agg_v, agg_hbm.at[pl.ds(nbase + blk * _NB, _NB)])
        return carry

    lax.fori_loop(0, _NBLK, blkbody, 0)


def _sc_aggregate(features, p, q, nbr_flat, node_flat):
    mesh = plsc.VectorSubcoreMesh(core_axis_name="c", subcore_axis_name="s")
    fn = functools.partial(
        pl.kernel,
        mesh=mesh,
        compiler_params=pltpu.CompilerParams(needs_layout_passes=False),
        out_type=jax.ShapeDtypeStruct((B, D), _F32),
        scratch_types=[
            pltpu.VMEM((_IPW,), jnp.int32),
            pltpu.VMEM((_IPW,), _F32),
            pltpu.VMEM((_BPW,), jnp.int32),
            pltpu.VMEM((_BPW,), _F32),
            pltpu.VMEM((_RB, D), _F32),
            pltpu.VMEM((_NB, D), _F32),
            pltpu.SemaphoreType.DMA,
            pltpu.SemaphoreType.DMA,
        ],
    )(_sc_body)
    return fn(features, p, q, nbr_flat, node_flat)


_MM_ROWS = 1024


def _mm_body(a_ref, k1_ref, nw_ref, o_ref):
    w = jnp.dot(k1_ref[...], nw_ref[...], preferred_element_type=_F32)
    o_ref[...] = jnp.dot(a_ref[...], w, preferred_element_type=_F32)


def _mm_pass(agg, k1, nw):
    return pl.pallas_call(
        _mm_body,
        grid=(B // _MM_ROWS,),
        in_specs=[
            pl.BlockSpec((_MM_ROWS, D), lambda i: (i, 0)),
            pl.BlockSpec((D, D), lambda i: (0, 0)),
            pl.BlockSpec((D, D), lambda i: (0, 0)),
        ],
        out_specs=pl.BlockSpec((_MM_ROWS, D), lambda i: (i, 0)),
        out_shape=jax.ShapeDtypeStruct((B, D), _F32),
    )(agg, k1, nw)


def kernel(features, node, neighbours, attention_weights, kernel, kernel1,
           neigh_weights):
    k0 = kernel.reshape(D, D)
    k1 = kernel1.reshape(D, D)
    p, q = _pq_pass(features, k0, k1, attention_weights)
    p = p.reshape(-1)
    q = q.reshape(-1)
    nbr_flat = neighbours.reshape(-1).astype(jnp.int32)
    node_flat = node.reshape(-1).astype(jnp.int32)
    agg = _sc_aggregate(features, p, q, nbr_flat, node_flat)
    return _mm_pass(agg, k1, neigh_weights)


# speedup vs baseline: 8.1048x; 1.7310x over previous
import functools

import jax
import jax.numpy as jnp
from jax import lax
from jax.experimental import pallas as pl
from jax.experimental.pallas import tpu as pltpu
from jax.experimental.pallas import tpu_sc as plsc

N_NODES = 100000
D = 128
B = 8192
K = 32

_F32 = jnp.float32


_PQ_ROWS = 2048
_PQ_PAD = _PQ_ROWS * ((N_NODES + _PQ_ROWS - 1) // _PQ_ROWS)


def _pq_body(f_ref, k0_ref, k1_ref, aw_ref, p_ref, q_ref, v12_ref):
    @pl.when(pl.program_id(0) == 0)
    def _():
        awn = aw_ref[0, :D].reshape(D, 1)
        awt = aw_ref[0, D:].reshape(D, 1)
        v1 = jnp.dot(k1_ref[...], awn, preferred_element_type=_F32)
        v2 = jnp.dot(k0_ref[...], awt, preferred_element_type=_F32)
        v12_ref[...] = jnp.concatenate(
            [v1, v2, jnp.zeros((D, 6), _F32)], axis=1)

    pq = jnp.dot(f_ref[...], v12_ref[...], preferred_element_type=_F32)
    t = jnp.transpose(pq.reshape(_PQ_ROWS // D, D, 8), (0, 2, 1))
    p_ref[...] = t[:, 0, :]
    q_ref[...] = t[:, 1, :]


def _pq_pass(features, k0, k1, aw):
    return pl.pallas_call(
        _pq_body,
        grid=(pl.cdiv(N_NODES, _PQ_ROWS),),
        in_specs=[
            pl.BlockSpec((_PQ_ROWS, D), lambda i: (i, 0)),
            pl.BlockSpec((D, D), lambda i: (0, 0)),
            pl.BlockSpec((D, D), lambda i: (0, 0)),
            pl.BlockSpec((1, 2 * D), lambda i: (0, 0)),
        ],
        out_specs=[
            pl.BlockSpec((_PQ_ROWS // D, D), lambda i: (i, 0)),
            pl.BlockSpec((_PQ_ROWS // D, D), lambda i: (i, 0)),
        ],
        out_shape=[
            jax.ShapeDtypeStruct((_PQ_PAD // D, D), _F32),
            jax.ShapeDtypeStruct((_PQ_PAD // D, D), _F32),
        ],
        scratch_shapes=[pltpu.VMEM((D, 8), _F32)],
    )(features, k0, k1, aw)


_NW = 32
_BPW = B // _NW
_IPW = _BPW * K
_NB = 8
_RB = _NB * K
_NBLK = _BPW // _NB
_C = D // 16


def _sc_body(feat, p_hbm, q_hbm, nbr_hbm, node_hbm, agg_hbm,
             idx_v, s_v, nidx_v, qv_v, rows_a, rows_b, agg_v,
             sem_a, sem_b, sem_p, sem_q):
    nc = plsc.get_sparse_core_info().num_cores
    wid = lax.axis_index("s") * nc + lax.axis_index("c")
    ibase = wid * _IPW
    nbase = wid * _BPW

    pltpu.sync_copy(nbr_hbm.at[pl.ds(ibase, _IPW)], idx_v)
    pltpu.sync_copy(node_hbm.at[pl.ds(nbase, _BPW)], nidx_v)
    pltpu.async_copy(feat.at[idx_v.at[pl.ds(0, _RB)]], rows_a, sem_a)
    pltpu.async_copy(feat.at[idx_v.at[pl.ds(_RB, _RB)]], rows_b, sem_b)
    pltpu.async_copy(p_hbm.at[idx_v], s_v, sem_p).wait()
    pltpu.async_copy(q_hbm.at[nidx_v], qv_v, sem_q).wait()

    def wbody(g, carry):
        qv = qv_v[pl.ds(g * 16, 16)]
        for j in range(16):
            base = (g * 16 + j) * K
            qb = qv[j]
            a0 = s_v[pl.ds(base, 16)] + qb
            a1 = s_v[pl.ds(base + 16, 16)] + qb
            a0 = jnp.where(a0 >= 0.0, a0, a0 * 0.2)
            a1 = jnp.where(a1 >= 0.0, a1, a1 * 0.2)
            m = jnp.maximum(jnp.max(a0), jnp.max(a1))
            e0 = jnp.exp(a0 - m)
            e1 = jnp.exp(a1 - m)
            den = jnp.broadcast_to(jnp.sum(e0) + jnp.sum(e1), (16,))
            s_v[pl.ds(base, 16)] = e0 / den
            s_v[pl.ds(base + 16, 16)] = e1 / den
        return carry

    lax.fori_loop(0, _BPW // 16, wbody, 0)

    def compute_block(blk, rows_v):
        def nbody(j, carry2):
            b0 = (blk * _NB + j) * K
            w0 = s_v[pl.ds(b0, 16)]
            w1 = s_v[pl.ds(b0 + 16, 16)]
            accs = tuple(jnp.zeros((16,), _F32) for _ in range(_C))
            for k in range(K):
                wk = w0[k] if k < 16 else w1[k - 16]
                r = j * K + k
                accs = tuple(
                    accs[c] + wk * rows_v[r, pl.ds(c * 16, 16)]
                    for c in range(_C))
            for c in range(_C):
                agg_v[j, pl.ds(c * 16, 16)] = accs[c]
            return carry2

        lax.fori_loop(0, _NB, nbody, 0)
        pltpu.sync_copy(agg_v, agg_hbm.at[pl.ds(nbase + blk * _NB, _NB)])

    def issue(blk, rows_v, sem):
        pltpu.async_copy(feat.at[idx_v.at[pl.ds(blk * _RB, _RB)]],
                         rows_v, sem)

    def wait(rows_v, sem):
        pltpu.make_async_copy(feat.at[idx_v.at[pl.ds(0, _RB)]],
                              rows_v, sem).wait()

    def pairbody(it, carry):
        blk = it * 2
        wait(rows_a, sem_a)
        compute_block(blk, rows_a)
        issue(blk + 2, rows_a, sem_a)
        wait(rows_b, sem_b)
        compute_block(blk + 1, rows_b)
        issue(blk + 3, rows_b, sem_b)
        return carry

    lax.fori_loop(0, _NBLK // 2 - 1, pairbody, 0)
    wait(rows_a, sem_a)
    compute_block(_NBLK - 2, rows_a)
    wait(rows_b, sem_b)
    compute_block(_NBLK - 1, rows_b)


def _sc_aggregate(features, p, q, nbr_flat, node_flat):
    mesh = plsc.VectorSubcoreMesh(core_axis_name="c", subcore_axis_name="s")
    fn = functools.partial(
        pl.kernel,
        mesh=mesh,
        compiler_params=pltpu.CompilerParams(needs_layout_passes=False),
        out_type=jax.ShapeDtypeStruct((B, D), _F32),
        scratch_types=[
            pltpu.VMEM((_IPW,), jnp.int32),
            pltpu.VMEM((_IPW,), _F32),
            pltpu.VMEM((_BPW,), jnp.int32),
            pltpu.VMEM((_BPW,), _F32),
            pltpu.VMEM((_RB, D), _F32),
            pltpu.VMEM((_RB, D), _F32),
            pltpu.VMEM((_NB, D), _F32),
            pltpu.SemaphoreType.DMA,
            pltpu.SemaphoreType.DMA,
            pltpu.SemaphoreType.DMA,
            pltpu.SemaphoreType.DMA,
        ],
    )(_sc_body)
    return fn(features, p, q, nbr_flat, node_flat)


_MM_ROWS = 1024


def _mm_body(a_ref, k1_ref, nw_ref, o_ref):
    w = jnp.dot(k1_ref[...], nw_ref[...], preferred_element_type=_F32)
    o_ref[...] = jnp.dot(a_ref[...], w, preferred_element_type=_F32)


def _mm_pass(agg, k1, nw):
    return pl.pallas_call(
        _mm_body,
        grid=(B // _MM_ROWS,),
        in_specs=[
            pl.BlockSpec((_MM_ROWS, D), lambda i: (i, 0)),
            pl.BlockSpec((D, D), lambda i: (0, 0)),
            pl.BlockSpec((D, D), lambda i: (0, 0)),
        ],
        out_specs=pl.BlockSpec((_MM_ROWS, D), lambda i: (i, 0)),
        out_shape=jax.ShapeDtypeStruct((B, D), _F32),
    )(agg, k1, nw)


def kernel(features, node, neighbours, attention_weights, kernel, kernel1,
           neigh_weights):
    k0 = kernel.reshape(D, D)
    k1 = kernel1.reshape(D, D)
    p, q = _pq_pass(features, k0, k1, attention_weights)
    p = p.reshape(-1)
    q = q.reshape(-1)
    nbr_flat = neighbours.reshape(-1).astype(jnp.int32)
    node_flat = node.reshape(-1).astype(jnp.int32)
    agg = _sc_aggregate(features, p, q, nbr_flat, node_flat)
    return _mm_pass(agg, k1, neigh_weights)


# speedup vs baseline: 8.1931x; 1.0109x over previous
import functools

import jax
import jax.numpy as jnp
from jax import lax
from jax.experimental import pallas as pl
from jax.experimental.pallas import tpu as pltpu
from jax.experimental.pallas import tpu_sc as plsc

N_NODES = 100000
D = 128
B = 8192
K = 32

_F32 = jnp.float32


_PQ_ROWS = 2048
_PQ_PAD = _PQ_ROWS * ((N_NODES + _PQ_ROWS - 1) // _PQ_ROWS)


def _pq_body(f_ref, k0_ref, k1_ref, aw_ref, p_ref, q_ref, v12_ref):
    @pl.when(pl.program_id(0) == 0)
    def _():
        awn = aw_ref[0, :D].reshape(D, 1)
        awt = aw_ref[0, D:].reshape(D, 1)
        v1 = jnp.dot(k1_ref[...], awn, preferred_element_type=_F32)
        v2 = jnp.dot(k0_ref[...], awt, preferred_element_type=_F32)
        v12_ref[...] = jnp.concatenate(
            [v1, v2, jnp.zeros((D, 6), _F32)], axis=1)

    pq = jnp.dot(f_ref[...].astype(jnp.bfloat16),
                 v12_ref[...].astype(jnp.bfloat16),
                 preferred_element_type=_F32)
    t = jnp.transpose(pq.reshape(_PQ_ROWS // D, D, 8), (0, 2, 1))
    p_ref[...] = t[:, 0, :]
    q_ref[...] = t[:, 1, :]


def _pq_pass(features, k0, k1, aw):
    return pl.pallas_call(
        _pq_body,
        grid=(pl.cdiv(N_NODES, _PQ_ROWS),),
        in_specs=[
            pl.BlockSpec((_PQ_ROWS, D), lambda i: (i, 0)),
            pl.BlockSpec((D, D), lambda i: (0, 0)),
            pl.BlockSpec((D, D), lambda i: (0, 0)),
            pl.BlockSpec((1, 2 * D), lambda i: (0, 0)),
        ],
        out_specs=[
            pl.BlockSpec((_PQ_ROWS // D, D), lambda i: (i, 0)),
            pl.BlockSpec((_PQ_ROWS // D, D), lambda i: (i, 0)),
        ],
        out_shape=[
            jax.ShapeDtypeStruct((_PQ_PAD // D, D), _F32),
            jax.ShapeDtypeStruct((_PQ_PAD // D, D), _F32),
        ],
        scratch_shapes=[pltpu.VMEM((D, 8), _F32)],
    )(features, k0, k1, aw)


_NW = 32
_BPW = B // _NW
_IPW = _BPW * K
_NB = 4
_RB = _NB * K
_NBLK = _BPW // _NB
_NBUF = 4
_C = D // 16


def _sc_body(feat, p_hbm, q_hbm, nbr_hbm, node_hbm, agg_hbm,
             idx_v, s_v, nidx_v, qv_v,
             rows_a, rows_b, rows_c, rows_d, agg_v,
             sem_a, sem_b, sem_c, sem_d, sem_p, sem_q):
    nc = plsc.get_sparse_core_info().num_cores
    wid = lax.axis_index("s") * nc + lax.axis_index("c")
    ibase = wid * _IPW
    nbase = wid * _BPW
    bufs = (rows_a, rows_b, rows_c, rows_d)
    sems = (sem_a, sem_b, sem_c, sem_d)

    pltpu.sync_copy(nbr_hbm.at[pl.ds(ibase, _IPW)], idx_v)
    pltpu.sync_copy(node_hbm.at[pl.ds(nbase, _BPW)], nidx_v)

    def issue(blk, rows_v, sem):
        pltpu.async_copy(feat.at[idx_v.at[pl.ds(blk * _RB, _RB)]],
                         rows_v, sem)

    def wait(rows_v, sem):
        pltpu.make_async_copy(feat.at[idx_v.at[pl.ds(0, _RB)]],
                              rows_v, sem).wait()

    for i in range(_NBUF):
        issue(i, bufs[i], sems[i])
    pltpu.async_copy(p_hbm.at[idx_v], s_v, sem_p).wait()
    pltpu.async_copy(q_hbm.at[nidx_v], qv_v, sem_q).wait()

    def wbody(g, carry):
        qv = qv_v[pl.ds(g * 16, 16)]
        for j in range(16):
            base = (g * 16 + j) * K
            qb = qv[j]
            a0 = s_v[pl.ds(base, 16)] + qb
            a1 = s_v[pl.ds(base + 16, 16)] + qb
            a0 = jnp.where(a0 >= 0.0, a0, a0 * 0.2)
            a1 = jnp.where(a1 >= 0.0, a1, a1 * 0.2)
            m = jnp.maximum(jnp.max(a0), jnp.max(a1))
            e0 = jnp.exp(a0 - m)
            e1 = jnp.exp(a1 - m)
            den = jnp.broadcast_to(jnp.sum(e0) + jnp.sum(e1), (16,))
            s_v[pl.ds(base, 16)] = e0 / den
            s_v[pl.ds(base + 16, 16)] = e1 / den
        return carry

    lax.fori_loop(0, _BPW // 16, wbody, 0)

    def compute_block(blk, rows_v):
        def nbody(j, carry2):
            b0 = (blk * _NB + j) * K
            w0 = s_v[pl.ds(b0, 16)]
            w1 = s_v[pl.ds(b0 + 16, 16)]
            accs = tuple(jnp.zeros((16,), _F32) for _ in range(_C))
            for k in range(K):
                wk = w0[k] if k < 16 else w1[k - 16]
                r = j * K + k
                accs = tuple(
                    accs[c] + wk * rows_v[r, pl.ds(c * 16, 16)]
                    for c in range(_C))
            for c in range(_C):
                agg_v[j, pl.ds(c * 16, 16)] = accs[c]
            return carry2

        lax.fori_loop(0, _NB, nbody, 0)
        pltpu.sync_copy(agg_v, agg_hbm.at[pl.ds(nbase + blk * _NB, _NB)])

    def ringbody(it, carry):
        blk = it * _NBUF
        for i in range(_NBUF):
            wait(bufs[i], sems[i])
            compute_block(blk + i, bufs[i])
            issue(blk + i + _NBUF, bufs[i], sems[i])
        return carry

    lax.fori_loop(0, _NBLK // _NBUF - 1, ringbody, 0)
    blk = _NBLK - _NBUF
    for i in range(_NBUF):
        wait(bufs[i], sems[i])
        compute_block(blk + i, bufs[i])


def _sc_aggregate(features, p, q, nbr_flat, node_flat):
    mesh = plsc.VectorSubcoreMesh(core_axis_name="c", subcore_axis_name="s")
    fn = functools.partial(
        pl.kernel,
        mesh=mesh,
        compiler_params=pltpu.CompilerParams(needs_layout_passes=False),
        out_type=jax.ShapeDtypeStruct((B, D), _F32),
        scratch_types=[
            pltpu.VMEM((_IPW,), jnp.int32),
            pltpu.VMEM((_IPW,), _F32),
            pltpu.VMEM((_BPW,), jnp.int32),
            pltpu.VMEM((_BPW,), _F32),
            pltpu.VMEM((_RB, D), _F32),
            pltpu.VMEM((_RB, D), _F32),
            pltpu.VMEM((_RB, D), _F32),
            pltpu.VMEM((_RB, D), _F32),
            pltpu.VMEM((_NB, D), _F32),
            pltpu.SemaphoreType.DMA,
            pltpu.SemaphoreType.DMA,
            pltpu.SemaphoreType.DMA,
            pltpu.SemaphoreType.DMA,
            pltpu.SemaphoreType.DMA,
            pltpu.SemaphoreType.DMA,
        ],
    )(_sc_body)
    return fn(features, p, q, nbr_flat, node_flat)


_MM_ROWS = 2048


def _mm_body(a_ref, k1_ref, nw_ref, o_ref):
    w = jnp.dot(k1_ref[...], nw_ref[...], preferred_element_type=_F32)
    o_ref[...] = jnp.dot(a_ref[...], w, preferred_element_type=_F32)


def _mm_pass(agg, k1, nw):
    return pl.pallas_call(
        _mm_body,
        grid=(B // _MM_ROWS,),
        in_specs=[
            pl.BlockSpec((_MM_ROWS, D), lambda i: (i, 0)),
            pl.BlockSpec((D, D), lambda i: (0, 0)),
            pl.BlockSpec((D, D), lambda i: (0, 0)),
        ],
        out_specs=pl.BlockSpec((_MM_ROWS, D), lambda i: (i, 0)),
        out_shape=jax.ShapeDtypeStruct((B, D), _F32),
    )(agg, k1, nw)


def kernel(features, node, neighbours, attention_weights, kernel, kernel1,
           neigh_weights):
    k0 = kernel.reshape(D, D)
    k1 = kernel1.reshape(D, D)
    p, q = _pq_pass(features, k0, k1, attention_weights)
    p = p.reshape(-1)
    q = q.reshape(-1)
    nbr_flat = neighbours.reshape(-1).astype(jnp.int32)
    node_flat = node.reshape(-1).astype(jnp.int32)
    agg = _sc_aggregate(features, p, q, nbr_flat, node_flat)
    return _mm_pass(agg, k1, neigh_weights)


# speedup vs baseline: 8.9394x; 1.0911x over previous
import functools

import jax
import jax.numpy as jnp
from jax import lax
from jax.experimental import pallas as pl
from jax.experimental.pallas import tpu as pltpu
from jax.experimental.pallas import tpu_sc as plsc

N_NODES = 100000
D = 128
B = 8192
K = 32

_F32 = jnp.float32


_PQ_ROWS = 4096
_PQ_PAD = _PQ_ROWS * ((N_NODES + _PQ_ROWS - 1) // _PQ_ROWS)


def _pq_body(f_ref, k0_ref, k1_ref, aw_ref, p_ref, q_ref, v12_ref):
    @pl.when(pl.program_id(0) == 0)
    def _():
        awn = aw_ref[0, :D].reshape(D, 1)
        awt = aw_ref[0, D:].reshape(D, 1)
        v1 = jnp.dot(k1_ref[...], awn, preferred_element_type=_F32)
        v2 = jnp.dot(k0_ref[...], awt, preferred_element_type=_F32)
        v12_ref[...] = jnp.concatenate(
            [v1, v2, jnp.zeros((D, 6), _F32)], axis=1)

    pq = jnp.dot(f_ref[...].astype(jnp.bfloat16),
                 v12_ref[...].astype(jnp.bfloat16),
                 preferred_element_type=_F32)
    t = jnp.transpose(pq.reshape(_PQ_ROWS // D, D, 8), (0, 2, 1))
    p_ref[...] = t[:, 0, :]
    q_ref[...] = t[:, 1, :]


def _pq_pass(features, k0, k1, aw):
    return pl.pallas_call(
        _pq_body,
        grid=(pl.cdiv(N_NODES, _PQ_ROWS),),
        in_specs=[
            pl.BlockSpec((_PQ_ROWS, D), lambda i: (i, 0)),
            pl.BlockSpec((D, D), lambda i: (0, 0)),
            pl.BlockSpec((D, D), lambda i: (0, 0)),
            pl.BlockSpec((1, 2 * D), lambda i: (0, 0)),
        ],
        out_specs=[
            pl.BlockSpec((_PQ_ROWS // D, D), lambda i: (i, 0)),
            pl.BlockSpec((_PQ_ROWS // D, D), lambda i: (i, 0)),
        ],
        out_shape=[
            jax.ShapeDtypeStruct((_PQ_PAD // D, D), _F32),
            jax.ShapeDtypeStruct((_PQ_PAD // D, D), _F32),
        ],
        scratch_shapes=[pltpu.VMEM((D, 8), _F32)],
    )(features, k0, k1, aw)


_NW = 32
_BPW = B // _NW
_IPW = _BPW * K
_NB = 4
_RB = _NB * K
_NBLK = _BPW // _NB
_NBUF = 4
_C = D // 16


def _sc_body(feat, p_hbm, q_hbm, nbr_hbm, node_hbm, agg_hbm,
             idx_v, s_v, nidx_v, qv_v,
             rows_a, rows_b, rows_c, rows_d, agg_v,
             sem_a, sem_b, sem_c, sem_d, sem_p, sem_q):
    nc = plsc.get_sparse_core_info().num_cores
    wid = lax.axis_index("s") * nc + lax.axis_index("c")
    ibase = wid * _IPW
    nbase = wid * _BPW
    bufs = (rows_a, rows_b, rows_c, rows_d)
    sems = (sem_a, sem_b, sem_c, sem_d)

    pltpu.sync_copy(nbr_hbm.at[pl.ds(ibase, _IPW)], idx_v)
    pltpu.sync_copy(node_hbm.at[pl.ds(nbase, _BPW)], nidx_v)

    def issue(blk, rows_v, sem):
        pltpu.async_copy(feat.at[idx_v.at[pl.ds(blk * _RB, _RB)]],
                         rows_v, sem)

    def wait(rows_v, sem):
        pltpu.make_async_copy(feat.at[idx_v.at[pl.ds(0, _RB)]],
                              rows_v, sem).wait()

    for i in range(_NBUF):
        issue(i, bufs[i], sems[i])
    chunk = _IPW // 4
    for ch in range(4):
        pltpu.async_copy(p_hbm.at[idx_v.at[pl.ds(ch * chunk, chunk)]],
                         s_v.at[pl.ds(ch * chunk, chunk)], sem_p)
    pltpu.async_copy(q_hbm.at[nidx_v], qv_v, sem_q).wait()

    def wbody(g, carry):
        qv = qv_v[pl.ds(g * 16, 16)]
        for j in range(16):
            base = (g * 16 + j) * K
            qb = qv[j]
            a0 = s_v[pl.ds(base, 16)] + qb
            a1 = s_v[pl.ds(base + 16, 16)] + qb
            a0 = jnp.where(a0 >= 0.0, a0, a0 * 0.2)
            a1 = jnp.where(a1 >= 0.0, a1, a1 * 0.2)
            m = jnp.max(jnp.maximum(a0, a1))
            e0 = jnp.exp(a0 - m)
            e1 = jnp.exp(a1 - m)
            den = jnp.broadcast_to(jnp.sum(e0 + e1), (16,))
            s_v[pl.ds(base, 16)] = e0 / den
            s_v[pl.ds(base + 16, 16)] = e1 / den
        return carry

    groups_per_chunk = _BPW // 16 // 4
    for ch in range(4):
        pltpu.make_async_copy(
            p_hbm.at[idx_v.at[pl.ds(0, chunk)]],
            s_v.at[pl.ds(ch * chunk, chunk)], sem_p).wait()
        lax.fori_loop(ch * groups_per_chunk, (ch + 1) * groups_per_chunk,
                      wbody, 0)

    def compute_block(blk, rows_v):
        def nbody(j, carry2):
            b0 = (blk * _NB + j) * K
            w0 = s_v[pl.ds(b0, 16)]
            w1 = s_v[pl.ds(b0 + 16, 16)]
            accs = tuple(jnp.zeros((16,), _F32) for _ in range(_C))
            for k in range(K):
                wk = w0[k] if k < 16 else w1[k - 16]
                r = j * K + k
                accs = tuple(
                    accs[c] + wk * rows_v[r, pl.ds(c * 16, 16)]
                    for c in range(_C))
            for c in range(_C):
                agg_v[j, pl.ds(c * 16, 16)] = accs[c]
            return carry2

        lax.fori_loop(0, _NB, nbody, 0)
        pltpu.sync_copy(agg_v, agg_hbm.at[pl.ds(nbase + blk * _NB, _NB)])

    def ringbody(it, carry):
        blk = it * _NBUF
        for i in range(_NBUF):
            wait(bufs[i], sems[i])
            compute_block(blk + i, bufs[i])
            issue(blk + i + _NBUF, bufs[i], sems[i])
        return carry

    lax.fori_loop(0, _NBLK // _NBUF - 1, ringbody, 0)
    blk = _NBLK - _NBUF
    for i in range(_NBUF):
        wait(bufs[i], sems[i])
        compute_block(blk + i, bufs[i])


def _sc_aggregate(features, p, q, nbr_flat, node_flat):
    mesh = plsc.VectorSubcoreMesh(core_axis_name="c", subcore_axis_name="s")
    fn = functools.partial(
        pl.kernel,
        mesh=mesh,
        compiler_params=pltpu.CompilerParams(needs_layout_passes=False),
        out_type=jax.ShapeDtypeStruct((B, D), _F32),
        scratch_types=[
            pltpu.VMEM((_IPW,), jnp.int32),
            pltpu.VMEM((_IPW,), _F32),
            pltpu.VMEM((_BPW,), jnp.int32),
            pltpu.VMEM((_BPW,), _F32),
            pltpu.VMEM((_RB, D), _F32),
            pltpu.VMEM((_RB, D), _F32),
            pltpu.VMEM((_RB, D), _F32),
            pltpu.VMEM((_RB, D), _F32),
            pltpu.VMEM((_NB, D), _F32),
            pltpu.SemaphoreType.DMA,
            pltpu.SemaphoreType.DMA,
            pltpu.SemaphoreType.DMA,
            pltpu.SemaphoreType.DMA,
            pltpu.SemaphoreType.DMA,
            pltpu.SemaphoreType.DMA,
        ],
    )(_sc_body)
    return fn(features, p, q, nbr_flat, node_flat)


_MM_ROWS = 2048


def _mm_body(a_ref, k1_ref, nw_ref, o_ref):
    w = jnp.dot(k1_ref[...], nw_ref[...], preferred_element_type=_F32)
    o_ref[...] = jnp.dot(a_ref[...], w, preferred_element_type=_F32)


def _mm_pass(agg, k1, nw):
    return pl.pallas_call(
        _mm_body,
        grid=(B // _MM_ROWS,),
        in_specs=[
            pl.BlockSpec((_MM_ROWS, D), lambda i: (i, 0)),
            pl.BlockSpec((D, D), lambda i: (0, 0)),
            pl.BlockSpec((D, D), lambda i: (0, 0)),
        ],
        out_specs=pl.BlockSpec((_MM_ROWS, D), lambda i: (i, 0)),
        out_shape=jax.ShapeDtypeStruct((B, D), _F32),
    )(agg, k1, nw)


def kernel(features, node, neighbours, attention_weights, kernel, kernel1,
           neigh_weights):
    k0 = kernel.reshape(D, D)
    k1 = kernel1.reshape(D, D)
    p, q = _pq_pass(features, k0, k1, attention_weights)
    p = p.reshape(-1)
    q = q.reshape(-1)
    nbr_flat = neighbours.reshape(-1).astype(jnp.int32)
    node_flat = node.reshape(-1).astype(jnp.int32)
    agg = _sc_aggregate(features, p, q, nbr_flat, node_flat)
    return _mm_pass(agg, k1, neigh_weights)
